# baseline (speedup 1.0000x reference)
import functools

import jax
import jax.numpy as jnp
from jax import lax
from jax.experimental import pallas as pl
from jax.experimental.pallas import tpu as pltpu
from jax.experimental.pallas import tpu_sc as plsc

VOCAB = 100000
HIDDEN = 128
B = 4096
S = 200
HALF = 100
NLANE = 16
NCOL = HIDDEN // NLANE
NBUF = 3


def _build(num_workers):
    n = B // num_workers
    mesh = plsc.VectorSubcoreMesh(core_axis_name="c", subcore_axis_name="s")

    @functools.partial(
        pl.kernel,
        mesh=mesh,
        out_type=jax.ShapeDtypeStruct((B, S, HIDDEN), jnp.float32),
        scratch_types=(
            [pltpu.VMEM((S,), jnp.int32) for _ in range(NBUF)]
            + [pltpu.VMEM((S, HIDDEN), jnp.float32) for _ in range(NBUF)]
            + [pltpu.VMEM((S, HIDDEN), jnp.float32)]
            + [pltpu.SemaphoreType.DMA for _ in range(3 * NBUF)]
        ),
    )
    def emb(ids_hbm, word_hbm, pos_hbm, out_hbm, *scratch):
        idx = scratch[:NBUF]
        rows = scratch[NBUF:2 * NBUF]
        pos_v = scratch[2 * NBUF]
        gsem = scratch[2 * NBUF + 1:2 * NBUF + 1 + NBUF]
        ssem = scratch[2 * NBUF + 1 + NBUF:2 * NBUF + 1 + 2 * NBUF]
        isem = scratch[2 * NBUF + 1 + 2 * NBUF:]

        cid = lax.axis_index("c")
        sid = lax.axis_index("s")
        wid = sid * 2 + cid
        base = wid * n

        pltpu.sync_copy(pos_hbm.at[pl.ds(0, S)], pos_v)

        splits = ((0, 128), (128, 72))

        def start_gathers(bb, c):
            for off, ln in splits:
                pltpu.async_copy(word_hbm.at[idx[bb].at[pl.ds(off, ln)]],
                                 rows[bb].at[pl.ds(off, ln)],
                                 gsem[bb])

        def wait_gathers(bb, c):
            for off, ln in splits:
                pltpu.make_async_copy(
                    word_hbm.at[idx[bb].at[pl.ds(off, ln)]],
                    rows[bb].at[pl.ds(off, ln)],
                    gsem[bb]).wait()

        pltpu.sync_copy(ids_hbm.at[base], idx[0])
        start_gathers(0, base)
        pltpu.sync_copy(ids_hbm.at[base + 1], idx[1])
        start_gathers(1, base + 1)

        def do_add(rows_b):
            def add_row(i, carry):
                for k in range(NCOL):
                    sl = pl.ds(k * NLANE, NLANE)
                    plsc.addupdate(rows_b.at[i, sl], pos_v[i, sl])
                return carry

            lax.fori_loop(0, S, add_row, 0)

        def when(pred, fn):
            if isinstance(pred, bool):
                if pred:
                    fn()
            else:
                pl.when(pred)(fn)

        def body(g, b):
            nb2 = (b + 2) % NBUF
            c = base + g
            has_next2 = g + 2 < n

            def start_idx_copy():
                pltpu.async_copy(ids_hbm.at[c + 2], idx[nb2], isem[nb2])

            when(has_next2, start_idx_copy)

            wait_gathers(b, c)

            def prefetch():
                def drain_scatter():
                    pltpu.make_async_copy(rows[nb2], out_hbm.at[c - 1],
                                          ssem[nb2]).wait()

                when(g >= 1, drain_scatter)
                pltpu.make_async_copy(ids_hbm.at[c + 2], idx[nb2],
                                      isem[nb2]).wait()
                start_gathers(nb2, c + 2)

            when(has_next2, prefetch)

            do_add(rows[b])
            pltpu.async_copy(rows[b], out_hbm.at[c], ssem[b])

        def tri(t, carry):
            for b in range(NBUF):
                body(NBUF * t + b, b)
            return carry

        ntri = n // NBUF
        lax.fori_loop(0, ntri, tri, 0)
        for j in range(ntri * NBUF, n):
            body(j, j % NBUF)

        for g in range(n - NBUF, n):
            pltpu.make_async_copy(rows[g % NBUF], out_hbm.at[base + g],
                                  ssem[g % NBUF]).wait()

    return emb


_emb_kernel = _build(32)


def kernel(input_ids, word_emb, pos_emb):
    return _emb_kernel(input_ids.astype(jnp.int32), word_emb, pos_emb)


# speedup vs baseline: 1.0794x; 1.0794x over previous
import functools

import jax
import jax.numpy as jnp
from jax import lax
from jax.experimental import pallas as pl
from jax.experimental.pallas import tpu as pltpu
from jax.experimental.pallas import tpu_sc as plsc

VOCAB = 100000
HIDDEN = 128
B = 4096
S = 200
HALF = 100
NLANE = 16
NCOL = HIDDEN // NLANE
NBUF = 3


def _build(num_workers):
    n = B // num_workers
    mesh = plsc.VectorSubcoreMesh(core_axis_name="c", subcore_axis_name="s")

    @functools.partial(
        pl.kernel,
        mesh=mesh,
        out_type=jax.ShapeDtypeStruct((B, S, HIDDEN), jnp.float32),
        scratch_types=(
            [pltpu.VMEM((S,), jnp.int32) for _ in range(NBUF)]
            + [pltpu.VMEM((S, HIDDEN), jnp.float32) for _ in range(NBUF)]
            + [pltpu.VMEM((S, HIDDEN), jnp.float32)]
            + [pltpu.SemaphoreType.DMA for _ in range(3 * NBUF)]
        ),
    )
    def emb(ids_hbm, word_hbm, pos_hbm, out_hbm, *scratch):
        idx = scratch[:NBUF]
        rows = scratch[NBUF:2 * NBUF]
        pos_v = scratch[2 * NBUF]
        gsem = scratch[2 * NBUF + 1:2 * NBUF + 1 + NBUF]
        ssem = scratch[2 * NBUF + 1 + NBUF:2 * NBUF + 1 + 2 * NBUF]
        isem = scratch[2 * NBUF + 1 + 2 * NBUF:]

        cid = lax.axis_index("c")
        sid = lax.axis_index("s")
        wid = sid * 2 + cid
        base = wid * n

        pltpu.sync_copy(pos_hbm.at[pl.ds(0, S)], pos_v)

        splits = ((0, 128), (128, 72))

        def start_gathers(bb, c):
            for off, ln in splits:
                pltpu.async_copy(word_hbm.at[idx[bb].at[pl.ds(off, ln)]],
                                 rows[bb].at[pl.ds(off, ln)],
                                 gsem[bb])

        def wait_gathers(bb, c):
            for off, ln in splits:
                pltpu.make_async_copy(
                    word_hbm.at[idx[bb].at[pl.ds(off, ln)]],
                    rows[bb].at[pl.ds(off, ln)],
                    gsem[bb]).wait()

        pltpu.sync_copy(ids_hbm.at[base], idx[0])
        start_gathers(0, base)

        def do_add(rows_b):
            def add_row(i, carry):
                for k in range(NCOL):
                    sl = pl.ds(k * NLANE, NLANE)
                    plsc.addupdate(rows_b.at[i, sl], pos_v[i, sl])
                return carry

            lax.fori_loop(0, S, add_row, 0)

        def when(pred, fn):
            if isinstance(pred, bool):
                if pred:
                    fn()
            else:
                pl.when(pred)(fn)

        def body(g, b):
            nb = (b + 1) % NBUF
            c = base + g
            has_next = g + 1 < n

            def start_idx_copy():
                pltpu.async_copy(ids_hbm.at[c + 1], idx[nb], isem[nb])

            when(has_next, start_idx_copy)

            wait_gathers(b, c)

            def prefetch():
                def drain_scatter():
                    pltpu.make_async_copy(rows[nb], out_hbm.at[c + 1 - NBUF],
                                          ssem[nb]).wait()

                when(g >= NBUF - 1, drain_scatter)
                pltpu.make_async_copy(ids_hbm.at[c + 1], idx[nb],
                                      isem[nb]).wait()
                start_gathers(nb, c + 1)

            when(has_next, prefetch)

            do_add(rows[b])
            pltpu.async_copy(rows[b], out_hbm.at[c], ssem[b])

        def tri(t, carry):
            for b in range(NBUF):
                body(NBUF * t + b, b)
            return carry

        ntri = n // NBUF
        lax.fori_loop(0, ntri, tri, 0)
        for j in range(ntri * NBUF, n):
            body(j, j % NBUF)

        for g in range(n - NBUF, n):
            pltpu.make_async_copy(rows[g % NBUF], out_hbm.at[base + g],
                                  ssem[g % NBUF]).wait()

    return emb


_emb_kernel = _build(32)


def kernel(input_ids, word_emb, pos_emb):
    return _emb_kernel(input_ids.astype(jnp.int32), word_emb, pos_emb)
